# baseline (speedup 1.0000x reference)
import jax
import jax.numpy as jnp
from jax import lax
from jax.experimental import pallas as pl
from jax.experimental.pallas import tpu as pltpu
from jax.experimental.pallas import tpu_sc as plsc

_N = 10000
_DH = 16
_TOPK = 5
_ALPHA = 0.5
_R = 200
_NSTRIP = _N // _R
_BSTRIPS = (4, 12, 12, 11, 11)

_NC = 2
_NS = 16
_NW = _NC * _NS
_G8 = 8
_CW = 1280
_NPH = 8
_COFF = tuple(i * _CW for i in range(_NPH))
_CLEN = tuple([_CW] * (_NPH - 1) + [_N - _CW * (_NPH - 1)])
_SLOT = tuple([c % 4 for c in range(_NPH - 1)] + [4])
_ARM = {0: (True, 5), 1: (True, 6), 2: (True, 3), 3: (False, 0),
        4: (False, 1), 5: (False, 2), 6: (True, 7)}
_MAX_GPW = max((s * _R // _G8) // _NW for s in _BSTRIPS) + 1


def _xtheta_body(x_ref, w_ref, b_ref, o_ref):
    z = lax.dot_general(
        x_ref[...], w_ref[...], (((1,), (1,)), ((), ())),
        preferred_element_type=jnp.float32)
    o_ref[...] = jnp.maximum(z + b_ref[...], 0.0)


def _tc_band_compute(xth_ref, a_ref, idx_ref, val_ref, strip_lo):
    r = strip_lo + pl.program_id(0)
    xr = xth_ref[pl.ds(r * _R, _R), :]
    z = lax.dot_general(
        xr, xth_ref[...], (((1,), (1,)), ((), ())),
        preferred_element_type=jnp.float32)
    a = jnp.tanh(z)
    a_ref[...] = a

    cols = lax.broadcasted_iota(jnp.int32, (_R, _N), 1).astype(jnp.float32)
    work = a
    sels = []
    tops = []
    for t in range(_TOPK):
        m = jnp.max(work, axis=1, keepdims=True)
        sel = jnp.min(
            jnp.where(work == m, cols, jnp.float32(2.0e9)),
            axis=1, keepdims=True)
        if t + 1 < _TOPK:
            work = jnp.where(cols == sel, jnp.float32(-2.0), work)
        sels.append(sel)
        tops.append(m)

    idx_ref[...] = jnp.concatenate(
        sels + [jnp.zeros((_R, 16 - _TOPK), jnp.float32)],
        axis=1).astype(jnp.int32)
    val_ref[...] = jnp.concatenate(
        [_ALPHA * m for m in tops]
        + [jnp.zeros((_R, 16 - _TOPK), jnp.float32)], axis=1)


def _make_tc_band(strip_lo, nstrips, first):
    band_rows = nstrips * _R

    if first:
        def body(xth_ref, a_ref, idx_ref, val_ref):
            _tc_band_compute(xth_ref, a_ref, idx_ref, val_ref, strip_lo)

        in_specs = [pl.BlockSpec((_N, _DH), lambda r: (0, 0))]
        aliases = {}
    else:
        def body(xth_ref, aprev_ref, a_ref, idx_ref, val_ref):
            del aprev_ref
            _tc_band_compute(xth_ref, a_ref, idx_ref, val_ref, strip_lo)

        in_specs = [
            pl.BlockSpec((_N, _DH), lambda r: (0, 0)),
            pl.BlockSpec((8, 128), lambda r: (0, 0)),
        ]
        aliases = {1: 0}

    return pl.pallas_call(
        body,
        grid=(nstrips,),
        in_specs=in_specs,
        out_specs=[
            pl.BlockSpec((_R, _N), lambda r, lo=strip_lo: (lo + r, 0)),
            pl.BlockSpec((_R, 16), lambda r: (r, 0)),
            pl.BlockSpec((_R, 16), lambda r: (r, 0)),
        ],
        out_shape=[
            jax.ShapeDtypeStruct((_N, _N), jnp.float32),
            jax.ShapeDtypeStruct((band_rows, 16), jnp.int32),
            jax.ShapeDtypeStruct((band_rows, 16), jnp.float32),
        ],
        input_output_aliases=aliases,
    )


def _sc_band_compute(adj_hbm, idx_hbm, val_hbm, p_hbm, bufs,
                     idx_all, val_all, sems_i, sems_o,
                     band_lo, n_band_groups):
    w = lax.axis_index("s") * _NC + lax.axis_index("c")
    gpw = n_band_groups // _NW
    xw = n_band_groups - gpw * _NW
    n_groups = gpw + (w < xw).astype(jnp.int32)
    g_lo = w * gpw + jnp.minimum(w, xw)
    lo = band_lo + g_lo * _G8
    lt = g_lo * _G8
    lane = lax.iota(jnp.int32, 16)
    mask5 = lane < _TOPK

    pltpu.sync_copy(idx_hbm.at[pl.ds(lt, gpw * _G8)],
                    idx_all.at[pl.ds(0, gpw * _G8)])
    pltpu.sync_copy(val_hbm.at[pl.ds(lt, gpw * _G8)],
                    val_all.at[pl.ds(0, gpw * _G8)])

    @pl.when(w < xw)
    def _():
        pltpu.sync_copy(idx_hbm.at[pl.ds(lt + gpw * _G8, _G8)],
                        idx_all.at[pl.ds(gpw * _G8, _G8)])
        pltpu.sync_copy(val_hbm.at[pl.ds(lt + gpw * _G8, _G8)],
                        val_all.at[pl.ds(gpw * _G8, _G8)])

    def in_copy(g, c):
        return pltpu.make_async_copy(
            adj_hbm.at[pl.ds(lo + g * _G8, _G8),
                       pl.ds(_COFF[c], _CLEN[c])],
            bufs[_SLOT[c]], sems_i[_SLOT[c]])

    def out_copy(g, c):
        return pltpu.make_async_copy(
            bufs[_SLOT[c]],
            p_hbm.at[pl.ds(lo + g * _G8, _G8),
                     pl.ds(_COFF[c], _CLEN[c])],
            sems_o[_SLOT[c]])

    def update(g, c):
        buf = bufs[_SLOT[c]]
        col_lo = _COFF[c]
        col_hi = col_lo + _CLEN[c]
        for j in range(_G8):
            idx_v = idx_all[g * _G8 + j]
            val_v = val_all[g * _G8 + j]
            m = jnp.logical_and(
                jnp.logical_and(mask5, idx_v >= col_lo), idx_v < col_hi)
            local = jnp.where(m, idx_v - col_lo, 0)
            rowv = jnp.full((16,), j, jnp.int32)
            plsc.addupdate_scatter(buf, [rowv, local], val_v, mask=m)

    in_copy(0, 0).start()

    def step(g, carry):
        for c in range(_NPH):
            in_copy(g, c).wait()
            update(g, c)
            if c == _NPH - 1:
                @pl.when(g + 1 < n_groups)
                def _():
                    out_copy(g, 4).wait()
                    in_copy(g + 1, 0).start()
            else:
                guarded, wp = _ARM[c]
                if guarded:
                    @pl.when(g > 0)
                    def _(wp=wp):
                        out_copy(g, wp).wait()
                else:
                    out_copy(g, wp).wait()
                in_copy(g, c + 1).start()

            out_copy(g, c).start()
        return carry

    lax.fori_loop(0, n_groups, step, 0)

    for pc in (3, 4, 5, 6, 7):
        out_copy(n_groups - 1, pc).wait()


_NSLOT = 5
_SC_SCRATCH = (
    [pltpu.VMEM((_G8, _CW), jnp.float32)] * (_NSLOT - 1)
    + [pltpu.VMEM((_G8, _CLEN[-1]), jnp.float32)]
    + [pltpu.VMEM((_MAX_GPW * _G8, 16), jnp.int32),
       pltpu.VMEM((_MAX_GPW * _G8, 16), jnp.float32)]
    + [pltpu.SemaphoreType.DMA] * (2 * _NSLOT)
)


def _make_sc_band(band_lo, n_band_groups, first):
    mesh = plsc.VectorSubcoreMesh(
        core_axis_name="c", subcore_axis_name="s",
        num_cores=_NC, num_subcores=_NS)

    def body(adj_hbm, idx_hbm, val_hbm, p_hbm, *rest):
        bufs = rest[:_NSLOT]
        ia, va = rest[_NSLOT], rest[_NSLOT + 1]
        sems = rest[_NSLOT + 2:]
        _sc_band_compute(adj_hbm, idx_hbm, val_hbm, p_hbm, bufs,
                         ia, va, sems[:_NSLOT], sems[_NSLOT:],
                         band_lo, n_band_groups)

    out_type = jax.ShapeDtypeStruct((_N, _N), jnp.float32) if first else ()
    return pl.kernel(
        body,
        out_type=out_type,
        mesh=mesh,
        compiler_params=pltpu.CompilerParams(needs_layout_passes=False),
        scratch_types=_SC_SCRATCH,
    )


@jax.jit
def kernel(X, adj, W_theta_w, W_theta_b):
    xth = pl.pallas_call(
        _xtheta_body,
        out_shape=jax.ShapeDtypeStruct((_N, _DH), jnp.float32),
    )(X, W_theta_w, W_theta_b.reshape(1, _DH))

    A = None
    pref = None
    strip_lo = 0
    for k, nstrips in enumerate(_BSTRIPS):
        band_lo = strip_lo * _R
        n_band_groups = nstrips * _R // _G8
        tc = _make_tc_band(strip_lo, nstrips, first=(k == 0))
        if k == 0:
            A, idx16, val16 = tc(xth)
            P0 = _make_sc_band(band_lo, n_band_groups, first=True)(
                adj, idx16, val16)
            pref = jax.new_ref(P0)
        else:
            A, idx16, val16 = tc(xth, A)
            _make_sc_band(band_lo, n_band_groups, first=False)(
                adj, idx16, val16, pref)
        strip_lo += nstrips

    P = pref[...]
    return P, A


# speedup vs baseline: 1.0210x; 1.0210x over previous
import jax
import jax.numpy as jnp
from jax import lax
from jax.experimental import pallas as pl
from jax.experimental.pallas import tpu as pltpu
from jax.experimental.pallas import tpu_sc as plsc

_N = 10000
_DH = 16
_TOPK = 5
_ALPHA = 0.5
_R = 200
_NSTRIP = _N // _R
_BSTRIPS = (6, 11, 11, 11, 11)

_NC = 2
_NS = 16
_NW = _NC * _NS
_G8 = 8
_CW = 1280
_NPH = 8
_COFF = tuple(i * _CW for i in range(_NPH))
_CLEN = tuple([_CW] * (_NPH - 1) + [_N - _CW * (_NPH - 1)])
_SLOT = tuple([c % 4 for c in range(_NPH - 1)] + [4])
_ARM = {0: (True, 5), 1: (True, 6), 2: (True, 3), 3: (False, 0),
        4: (False, 1), 5: (False, 2), 6: (True, 7)}
_MAX_GPW = max((s * _R // _G8) // _NW for s in _BSTRIPS) + 1


def _xtheta_body(x_ref, w_ref, b_ref, o_ref):
    z = lax.dot_general(
        x_ref[...], w_ref[...], (((1,), (1,)), ((), ())),
        preferred_element_type=jnp.float32)
    o_ref[...] = jnp.maximum(z + b_ref[...], 0.0)


def _tc_band_compute(xth_ref, a_ref, idx_ref, val_ref, strip_lo):
    r = strip_lo + pl.program_id(0)
    xr = xth_ref[pl.ds(r * _R, _R), :]
    z = lax.dot_general(
        xr, xth_ref[...], (((1,), (1,)), ((), ())),
        preferred_element_type=jnp.float32)
    a = jnp.tanh(z)
    a_ref[...] = a

    cols = lax.broadcasted_iota(jnp.int32, (_R, _N), 1).astype(jnp.float32)
    work = a
    sels = []
    tops = []
    for t in range(_TOPK):
        m = jnp.max(work, axis=1, keepdims=True)
        sel = jnp.min(
            jnp.where(work == m, cols, jnp.float32(2.0e9)),
            axis=1, keepdims=True)
        if t + 1 < _TOPK:
            work = jnp.where(cols == sel, jnp.float32(-2.0), work)
        sels.append(sel)
        tops.append(m)

    idx_ref[...] = jnp.concatenate(
        sels + [jnp.zeros((_R, 16 - _TOPK), jnp.float32)],
        axis=1).astype(jnp.int32)
    val_ref[...] = jnp.concatenate(
        [_ALPHA * m for m in tops]
        + [jnp.zeros((_R, 16 - _TOPK), jnp.float32)], axis=1)


def _make_tc_band(strip_lo, nstrips, first):
    band_rows = nstrips * _R

    if first:
        def body(xth_ref, a_ref, idx_ref, val_ref):
            _tc_band_compute(xth_ref, a_ref, idx_ref, val_ref, strip_lo)

        in_specs = [pl.BlockSpec((_N, _DH), lambda r: (0, 0))]
        aliases = {}
    else:
        def body(xth_ref, aprev_ref, a_ref, idx_ref, val_ref):
            del aprev_ref
            _tc_band_compute(xth_ref, a_ref, idx_ref, val_ref, strip_lo)

        in_specs = [
            pl.BlockSpec((_N, _DH), lambda r: (0, 0)),
            pl.BlockSpec((8, 128), lambda r: (0, 0)),
        ]
        aliases = {1: 0}

    return pl.pallas_call(
        body,
        grid=(nstrips,),
        in_specs=in_specs,
        out_specs=[
            pl.BlockSpec((_R, _N), lambda r, lo=strip_lo: (lo + r, 0)),
            pl.BlockSpec((_R, 16), lambda r: (r, 0)),
            pl.BlockSpec((_R, 16), lambda r: (r, 0)),
        ],
        out_shape=[
            jax.ShapeDtypeStruct((_N, _N), jnp.float32),
            jax.ShapeDtypeStruct((band_rows, 16), jnp.int32),
            jax.ShapeDtypeStruct((band_rows, 16), jnp.float32),
        ],
        input_output_aliases=aliases,
    )


def _sc_band_compute(adj_hbm, idx_hbm, val_hbm, p_hbm, bufs,
                     idx_all, val_all, sems_i, sems_o,
                     band_lo, n_band_groups):
    w = lax.axis_index("s") * _NC + lax.axis_index("c")
    gpw = n_band_groups // _NW
    xw = n_band_groups - gpw * _NW
    n_groups = gpw + (w < xw).astype(jnp.int32)
    g_lo = w * gpw + jnp.minimum(w, xw)
    lo = band_lo + g_lo * _G8
    lt = g_lo * _G8
    lane = lax.iota(jnp.int32, 16)
    mask5 = lane < _TOPK

    pltpu.sync_copy(idx_hbm.at[pl.ds(lt, gpw * _G8)],
                    idx_all.at[pl.ds(0, gpw * _G8)])
    pltpu.sync_copy(val_hbm.at[pl.ds(lt, gpw * _G8)],
                    val_all.at[pl.ds(0, gpw * _G8)])

    @pl.when(w < xw)
    def _():
        pltpu.sync_copy(idx_hbm.at[pl.ds(lt + gpw * _G8, _G8)],
                        idx_all.at[pl.ds(gpw * _G8, _G8)])
        pltpu.sync_copy(val_hbm.at[pl.ds(lt + gpw * _G8, _G8)],
                        val_all.at[pl.ds(gpw * _G8, _G8)])

    def in_copy(g, c):
        return pltpu.make_async_copy(
            adj_hbm.at[pl.ds(lo + g * _G8, _G8),
                       pl.ds(_COFF[c], _CLEN[c])],
            bufs[_SLOT[c]], sems_i[_SLOT[c]])

    def out_copy(g, c):
        return pltpu.make_async_copy(
            bufs[_SLOT[c]],
            p_hbm.at[pl.ds(lo + g * _G8, _G8),
                     pl.ds(_COFF[c], _CLEN[c])],
            sems_o[_SLOT[c]])

    def update(g, c):
        buf = bufs[_SLOT[c]]
        col_lo = _COFF[c]
        col_hi = col_lo + _CLEN[c]
        for j in range(_G8):
            idx_v = idx_all[g * _G8 + j]
            val_v = val_all[g * _G8 + j]
            m = jnp.logical_and(
                jnp.logical_and(mask5, idx_v >= col_lo), idx_v < col_hi)
            local = jnp.where(m, idx_v - col_lo, 0)
            rowv = jnp.full((16,), j, jnp.int32)
            plsc.addupdate_scatter(buf, [rowv, local], val_v, mask=m)

    in_copy(0, 0).start()

    def step(g, carry):
        for c in range(_NPH):
            in_copy(g, c).wait()
            update(g, c)
            if c == _NPH - 1:
                @pl.when(g + 1 < n_groups)
                def _():
                    out_copy(g, 4).wait()
                    in_copy(g + 1, 0).start()
            else:
                guarded, wp = _ARM[c]
                if guarded:
                    @pl.when(g > 0)
                    def _(wp=wp):
                        out_copy(g, wp).wait()
                else:
                    out_copy(g, wp).wait()
                in_copy(g, c + 1).start()

            out_copy(g, c).start()
        return carry

    lax.fori_loop(0, n_groups, step, 0)

    for pc in (3, 4, 5, 6, 7):
        out_copy(n_groups - 1, pc).wait()


_NSLOT = 5
_SC_SCRATCH = (
    [pltpu.VMEM((_G8, _CW), jnp.float32)] * (_NSLOT - 1)
    + [pltpu.VMEM((_G8, _CLEN[-1]), jnp.float32)]
    + [pltpu.VMEM((_MAX_GPW * _G8, 16), jnp.int32),
       pltpu.VMEM((_MAX_GPW * _G8, 16), jnp.float32)]
    + [pltpu.SemaphoreType.DMA] * (2 * _NSLOT)
)


def _make_sc_band(band_lo, n_band_groups, first):
    mesh = plsc.VectorSubcoreMesh(
        core_axis_name="c", subcore_axis_name="s",
        num_cores=_NC, num_subcores=_NS)

    def body(adj_hbm, idx_hbm, val_hbm, p_hbm, *rest):
        bufs = rest[:_NSLOT]
        ia, va = rest[_NSLOT], rest[_NSLOT + 1]
        sems = rest[_NSLOT + 2:]
        _sc_band_compute(adj_hbm, idx_hbm, val_hbm, p_hbm, bufs,
                         ia, va, sems[:_NSLOT], sems[_NSLOT:],
                         band_lo, n_band_groups)

    out_type = jax.ShapeDtypeStruct((_N, _N), jnp.float32) if first else ()
    return pl.kernel(
        body,
        out_type=out_type,
        mesh=mesh,
        compiler_params=pltpu.CompilerParams(needs_layout_passes=False),
        scratch_types=_SC_SCRATCH,
    )


@jax.jit
def kernel(X, adj, W_theta_w, W_theta_b):
    xth = pl.pallas_call(
        _xtheta_body,
        out_shape=jax.ShapeDtypeStruct((_N, _DH), jnp.float32),
    )(X, W_theta_w, W_theta_b.reshape(1, _DH))

    A = None
    pref = None
    strip_lo = 0
    for k, nstrips in enumerate(_BSTRIPS):
        band_lo = strip_lo * _R
        n_band_groups = nstrips * _R // _G8
        tc = _make_tc_band(strip_lo, nstrips, first=(k == 0))
        if k == 0:
            A, idx16, val16 = tc(xth)
            P0 = _make_sc_band(band_lo, n_band_groups, first=True)(
                adj, idx16, val16)
            pref = jax.new_ref(P0)
        else:
            A, idx16, val16 = tc(xth, A)
            _make_sc_band(band_lo, n_band_groups, first=False)(
                adj, idx16, val16, pref)
        strip_lo += nstrips

    P = pref[...]
    return P, A
